# baseline (speedup 1.0000x reference)
import jax
import jax.numpy as jnp
from jax.experimental import pallas as pl


def _add_kernel(x_ref, pos_ref, o_ref):
    o_ref[...] = x_ref[...] + pos_ref[...]


def _part(x, positions, blk=512):
    batch, seq_len, d_model = x.shape
    n_seq = seq_len // blk
    return pl.pallas_call(
        _add_kernel,
        grid=(n_seq,),
        in_specs=[
            pl.BlockSpec((batch, blk, d_model), lambda i: (0, i, 0)),
            pl.BlockSpec((blk, d_model), lambda i: (i, 0)),
        ],
        out_specs=pl.BlockSpec((batch, blk, d_model), lambda i: (0, i, 0)),
        out_shape=jax.ShapeDtypeStruct(x.shape, x.dtype),
    )(x, positions)


def kernel(inputs, position_embedding):
    batch, seq_len, d_model = inputs.shape
    positions = position_embedding[:seq_len, :]
    a = _part(inputs[:3], positions)
    b = _part(inputs[3:], positions)
    return jnp.concatenate([a, b], axis=0)
